# baseline (speedup 1.0000x reference)
import jax
import jax.numpy as jnp
from jax.experimental import pallas as pl
from jax.experimental.pallas import tpu as pltpu

VOCAB, EMBED, HIDDEN = 100000, 256, 512
B, T = 16, 20
BT = B * T
VTILE = 2048
LOOKAHEAD = 2

_NT = (((1,), (1,)), ((), ()))


def _gru_proj_body(idx_ref, e_any, enc_ref, wih0_ref, whh0_ref, bi0_ref,
                   bh0_ref, wih1_ref, whh1_ref, bi1_ref, bh1_ref,
                   w1_ref, b1_ref, w2_ref, b2_ref, out_ref,
                   emb_ref, ys_ref, sem):
    def row_copy(t, b):
        idx = idx_ref[b, t]
        return pltpu.make_async_copy(
            e_any.at[pl.ds(idx, 1), :],
            emb_ref.at[pl.ds(t * B + b, 1), :],
            sem)

    def issue_step(t):
        for b in range(B):
            row_copy(t, b).start()

    def wait_step(t):
        for b in range(B):
            row_copy(t, b).wait()

    for t in range(LOOKAHEAD):
        issue_step(t)

    def gru(x, h, wih, whh, bi, bh):
        gi = jax.lax.dot_general(x, wih, _NT,
                                 preferred_element_type=jnp.float32) + bi
        gh = jax.lax.dot_general(h, whh, _NT,
                                 preferred_element_type=jnp.float32) + bh
        i_r, i_z, i_n = gi[:, :HIDDEN], gi[:, HIDDEN:2 * HIDDEN], gi[:, 2 * HIDDEN:]
        h_r, h_z, h_n = gh[:, :HIDDEN], gh[:, HIDDEN:2 * HIDDEN], gh[:, 2 * HIDDEN:]
        r = jax.nn.sigmoid(i_r + h_r)
        z = jax.nn.sigmoid(i_z + h_z)
        n = jnp.tanh(i_n + r * h_n)
        return (1.0 - z) * n + z * h

    def step(t, carry):
        h0, h1 = carry

        @pl.when(t < T - LOOKAHEAD)
        def _():
            issue_step(t + LOOKAHEAD)

        wait_step(t)
        x = emb_ref[pl.ds(t * B, B), :]
        h0n = gru(x, h0, wih0_ref[...], whh0_ref[...], bi0_ref[...], bh0_ref[...])
        h1n = gru(h0n, h1, wih1_ref[...], whh1_ref[...], bi1_ref[...], bh1_ref[...])
        ys_ref[pl.ds(t * B, B), :] = h1n
        return (h0n, h1n)

    h0 = enc_ref[0]
    h1 = enc_ref[1]
    jax.lax.fori_loop(0, T, step, (h0, h1))
    ys = ys_ref[...]
    hid = jnp.tanh(jax.lax.dot_general(ys, w1_ref[...], _NT,
                                       preferred_element_type=jnp.float32)
                   + b1_ref[...])
    out_ref[...] = jax.lax.dot_general(hid, w2_ref[...], _NT,
                                       preferred_element_type=jnp.float32) + b2_ref[...]


def _logits_body(proj_ref, e_ref, gb_ref, out_ref):
    res = jax.lax.dot_general(
        proj_ref[...], e_ref[...], _NT,
        preferred_element_type=jnp.float32) + gb_ref[...]
    out_ref[...] = res.reshape(T, B, res.shape[-1])


def kernel(encoding, trg, E, W_ih0, W_hh0, b_ih0, b_hh0, W_ih1, W_hh1,
           b_ih1, b_hh1, W1, b1, W2, b2, g_b):
    idx = trg.astype(jnp.int32)

    gru_weights = (
        encoding,
        W_ih0, W_hh0, b_ih0.reshape(1, -1), b_hh0.reshape(1, -1),
        W_ih1, W_hh1, b_ih1.reshape(1, -1), b_hh1.reshape(1, -1),
        W1, b1.reshape(1, -1), W2, b2.reshape(1, -1),
    )

    proj = pl.pallas_call(
        _gru_proj_body,
        in_specs=[pl.BlockSpec(memory_space=pltpu.SMEM),
                  pl.BlockSpec(memory_space=pltpu.MemorySpace.HBM)]
                 + [pl.BlockSpec(memory_space=pltpu.MemorySpace.VMEM)] * 13,
        out_shape=jax.ShapeDtypeStruct((BT, EMBED), jnp.float32),
        scratch_shapes=[pltpu.VMEM((BT, EMBED), jnp.float32),
                        pltpu.VMEM((BT, HIDDEN), jnp.float32),
                        pltpu.SemaphoreType.DMA],
    )(idx, E, *gru_weights)

    nv = pl.cdiv(VOCAB, VTILE)
    logits_tb = pl.pallas_call(
        _logits_body,
        grid=(nv,),
        in_specs=[
            pl.BlockSpec((BT, EMBED), lambda i: (0, 0)),
            pl.BlockSpec((VTILE, EMBED), lambda i: (i, 0)),
            pl.BlockSpec((1, VTILE), lambda i: (0, i)),
        ],
        out_specs=pl.BlockSpec((T, B, VTILE), lambda i: (0, 0, i)),
        out_shape=jax.ShapeDtypeStruct((T, B, VOCAB), jnp.float32),
        compiler_params=pltpu.CompilerParams(
            dimension_semantics=("arbitrary",)),
    )(proj, E, g_b.reshape(1, VOCAB))

    return logits_tb.transpose(1, 0, 2)


# speedup vs baseline: 1.1020x; 1.1020x over previous
import jax
import jax.numpy as jnp
from jax.experimental import pallas as pl
from jax.experimental.pallas import tpu as pltpu

VOCAB, EMBED, HIDDEN = 100000, 256, 512
B, T = 16, 20
BT = B * T
VTILE = 10240
LOOKAHEAD = 2

_NT = (((1,), (1,)), ((), ()))


def _gru_proj_body(idx_ref, e_any, enc_ref, wih0_ref, whh0_ref, bi0_ref,
                   bh0_ref, wih1_ref, whh1_ref, bi1_ref, bh1_ref,
                   w1_ref, b1_ref, w2_ref, b2_ref, out_ref,
                   emb_ref, ys_ref, sem):
    def row_copy(t, b):
        idx = idx_ref[b, t]
        return pltpu.make_async_copy(
            e_any.at[pl.ds(idx, 1), :],
            emb_ref.at[pl.ds(t * B + b, 1), :],
            sem)

    def issue_step(t):
        for b in range(B):
            row_copy(t, b).start()

    def wait_step(t):
        for b in range(B):
            row_copy(t, b).wait()

    for t in range(LOOKAHEAD):
        issue_step(t)

    def gru(x, h, wih, whh, bi, bh):
        gi = jax.lax.dot_general(x, wih, _NT,
                                 preferred_element_type=jnp.float32) + bi
        gh = jax.lax.dot_general(h, whh, _NT,
                                 preferred_element_type=jnp.float32) + bh
        i_r, i_z, i_n = gi[:, :HIDDEN], gi[:, HIDDEN:2 * HIDDEN], gi[:, 2 * HIDDEN:]
        h_r, h_z, h_n = gh[:, :HIDDEN], gh[:, HIDDEN:2 * HIDDEN], gh[:, 2 * HIDDEN:]
        r = jax.nn.sigmoid(i_r + h_r)
        z = jax.nn.sigmoid(i_z + h_z)
        n = jnp.tanh(i_n + r * h_n)
        return (1.0 - z) * n + z * h

    def step(t, carry):
        h0, h1 = carry

        @pl.when(t < T - LOOKAHEAD)
        def _():
            issue_step(t + LOOKAHEAD)

        wait_step(t)
        x = emb_ref[pl.ds(t * B, B), :]
        h0n = gru(x, h0, wih0_ref[...], whh0_ref[...], bi0_ref[...], bh0_ref[...])
        h1n = gru(h0n, h1, wih1_ref[...], whh1_ref[...], bi1_ref[...], bh1_ref[...])
        ys_ref[pl.ds(t * B, B), :] = h1n
        return (h0n, h1n)

    h0 = enc_ref[0]
    h1 = enc_ref[1]
    jax.lax.fori_loop(0, T, step, (h0, h1))
    ys = ys_ref[...]
    hid = jnp.tanh(jax.lax.dot_general(ys, w1_ref[...], _NT,
                                       preferred_element_type=jnp.float32)
                   + b1_ref[...])
    out_ref[...] = jax.lax.dot_general(hid, w2_ref[...], _NT,
                                       preferred_element_type=jnp.float32) + b2_ref[...]


def _logits_body(proj_ref, e_ref, gb_ref, out_ref):
    res = jax.lax.dot_general(
        proj_ref[...], e_ref[...], _NT,
        preferred_element_type=jnp.float32) + gb_ref[...]
    out_ref[...] = res.reshape(T, B, res.shape[-1])


def kernel(encoding, trg, E, W_ih0, W_hh0, b_ih0, b_hh0, W_ih1, W_hh1,
           b_ih1, b_hh1, W1, b1, W2, b2, g_b):
    idx = trg.astype(jnp.int32)

    gru_weights = (
        encoding,
        W_ih0, W_hh0, b_ih0.reshape(1, -1), b_hh0.reshape(1, -1),
        W_ih1, W_hh1, b_ih1.reshape(1, -1), b_hh1.reshape(1, -1),
        W1, b1.reshape(1, -1), W2, b2.reshape(1, -1),
    )

    proj = pl.pallas_call(
        _gru_proj_body,
        in_specs=[pl.BlockSpec(memory_space=pltpu.SMEM),
                  pl.BlockSpec(memory_space=pltpu.MemorySpace.HBM)]
                 + [pl.BlockSpec(memory_space=pltpu.MemorySpace.VMEM)] * 13,
        out_shape=jax.ShapeDtypeStruct((BT, EMBED), jnp.float32),
        scratch_shapes=[pltpu.VMEM((BT, EMBED), jnp.float32),
                        pltpu.VMEM((BT, HIDDEN), jnp.float32),
                        pltpu.SemaphoreType.DMA],
    )(idx, E, *gru_weights)

    nv = pl.cdiv(VOCAB, VTILE)
    logits_tb = pl.pallas_call(
        _logits_body,
        grid=(nv,),
        in_specs=[
            pl.BlockSpec((BT, EMBED), lambda i: (0, 0)),
            pl.BlockSpec((VTILE, EMBED), lambda i: (i, 0)),
            pl.BlockSpec((1, VTILE), lambda i: (0, i)),
        ],
        out_specs=pl.BlockSpec((T, B, VTILE), lambda i: (0, 0, i)),
        out_shape=jax.ShapeDtypeStruct((T, B, VOCAB), jnp.float32),
        compiler_params=pltpu.CompilerParams(
            dimension_semantics=("arbitrary",)),
    )(proj, E, g_b.reshape(1, VOCAB))

    return logits_tb.transpose(1, 0, 2)


# speedup vs baseline: 1.2766x; 1.1585x over previous
import jax
import jax.numpy as jnp
from jax.experimental import pallas as pl
from jax.experimental.pallas import tpu as pltpu

VOCAB, EMBED, HIDDEN = 100000, 256, 512
B, T = 16, 20
BT = B * T
VTILE = 10240
LOOKAHEAD = 2

_NT = (((1,), (1,)), ((), ()))


def _gru_proj_body(idx_ref, e_any, enc_ref, wih0_ref, whh0_ref, bi0_ref,
                   bh0_ref, wih1_ref, whh1_ref, bi1_ref, bh1_ref,
                   w1_ref, b1_ref, w2_ref, b2_ref, out_ref,
                   emb_ref, ys_ref, wih0t_ref, whh0t_ref, wih1t_ref,
                   whh1t_ref, sem):
    def row_copy(t, b):
        idx = idx_ref[b, t]
        return pltpu.make_async_copy(
            e_any.at[pl.ds(idx, 1), :],
            emb_ref.at[pl.ds(t * B + b, 1), :],
            sem)

    def issue_step(t):
        for b in range(B):
            row_copy(t, b).start()

    def wait_step(t):
        for b in range(B):
            row_copy(t, b).wait()

    for t in range(LOOKAHEAD):
        issue_step(t)

    wih0t_ref[...] = wih0_ref[...].T
    whh0t_ref[...] = whh0_ref[...].T
    wih1t_ref[...] = wih1_ref[...].T
    whh1t_ref[...] = whh1_ref[...].T

    def gru(x, h, wih, whh, bi, bh):
        gi = jnp.dot(x, wih, preferred_element_type=jnp.float32) + bi
        gh = jnp.dot(h, whh, preferred_element_type=jnp.float32) + bh
        i_r, i_z, i_n = gi[:, :HIDDEN], gi[:, HIDDEN:2 * HIDDEN], gi[:, 2 * HIDDEN:]
        h_r, h_z, h_n = gh[:, :HIDDEN], gh[:, HIDDEN:2 * HIDDEN], gh[:, 2 * HIDDEN:]
        r = jax.nn.sigmoid(i_r + h_r)
        z = jax.nn.sigmoid(i_z + h_z)
        n = jnp.tanh(i_n + r * h_n)
        return (1.0 - z) * n + z * h

    def step(t, carry):
        h0, h1 = carry

        @pl.when(t < T - LOOKAHEAD)
        def _():
            issue_step(t + LOOKAHEAD)

        wait_step(t)
        x = emb_ref[pl.ds(t * B, B), :]
        h0n = gru(x, h0, wih0t_ref[...], whh0t_ref[...],
                  bi0_ref[...].reshape(1, -1), bh0_ref[...].reshape(1, -1))
        h1n = gru(h0n, h1, wih1t_ref[...], whh1t_ref[...],
                  bi1_ref[...].reshape(1, -1), bh1_ref[...].reshape(1, -1))
        ys_ref[pl.ds(t * B, B), :] = h1n
        return (h0n, h1n)

    h0 = enc_ref[0]
    h1 = enc_ref[1]
    jax.lax.fori_loop(0, T, step, (h0, h1))
    ys = ys_ref[...]
    hid = jnp.tanh(jax.lax.dot_general(ys, w1_ref[...], _NT,
                                       preferred_element_type=jnp.float32)
                   + b1_ref[...].reshape(1, -1))
    out_ref[...] = jax.lax.dot_general(hid, w2_ref[...], _NT,
                                       preferred_element_type=jnp.float32) \
        + b2_ref[...].reshape(1, -1)


def _logits_body(proj_ref, e_ref, gb_ref, out_ref):
    res = jax.lax.dot_general(
        proj_ref[...], e_ref[...], _NT,
        preferred_element_type=jnp.float32) + gb_ref[...]
    out_ref[...] = res.reshape(T, B, res.shape[-1])


def kernel(encoding, trg, E, W_ih0, W_hh0, b_ih0, b_hh0, W_ih1, W_hh1,
           b_ih1, b_hh1, W1, b1, W2, b2, g_b):
    idx = trg.astype(jnp.int32)

    gru_weights = (
        encoding,
        W_ih0, W_hh0, b_ih0, b_hh0,
        W_ih1, W_hh1, b_ih1, b_hh1,
        W1, b1, W2, b2,
    )

    proj = pl.pallas_call(
        _gru_proj_body,
        in_specs=[pl.BlockSpec(memory_space=pltpu.SMEM),
                  pl.BlockSpec(memory_space=pltpu.MemorySpace.HBM)]
                 + [pl.BlockSpec(memory_space=pltpu.MemorySpace.VMEM)] * 13,
        out_shape=jax.ShapeDtypeStruct((BT, EMBED), jnp.float32),
        scratch_shapes=[pltpu.VMEM((BT, EMBED), jnp.float32),
                        pltpu.VMEM((BT, HIDDEN), jnp.float32),
                        pltpu.VMEM((EMBED, 3 * HIDDEN), jnp.float32),
                        pltpu.VMEM((HIDDEN, 3 * HIDDEN), jnp.float32),
                        pltpu.VMEM((HIDDEN, 3 * HIDDEN), jnp.float32),
                        pltpu.VMEM((HIDDEN, 3 * HIDDEN), jnp.float32),
                        pltpu.SemaphoreType.DMA],
    )(idx, E, *gru_weights)

    nv = pl.cdiv(VOCAB, VTILE)
    logits_tb = pl.pallas_call(
        _logits_body,
        grid=(nv,),
        in_specs=[
            pl.BlockSpec((BT, EMBED), lambda i: (0, 0)),
            pl.BlockSpec((VTILE, EMBED), lambda i: (i, 0)),
            pl.BlockSpec((1, VTILE), lambda i: (0, i)),
        ],
        out_specs=pl.BlockSpec((T, B, VTILE), lambda i: (0, 0, i)),
        out_shape=jax.ShapeDtypeStruct((T, B, VOCAB), jnp.float32),
        compiler_params=pltpu.CompilerParams(
            dimension_semantics=("arbitrary",)),
    )(proj, E, g_b.reshape(1, VOCAB))

    return logits_tb.transpose(1, 0, 2)


# speedup vs baseline: 1.3000x; 1.0183x over previous
import jax
import jax.numpy as jnp
from jax.experimental import pallas as pl
from jax.experimental.pallas import tpu as pltpu

VOCAB, EMBED, HIDDEN = 100000, 256, 512
B, T = 16, 20
BT = B * T
VTILE = 10240
LOOKAHEAD = 2

_NT = (((1,), (1,)), ((), ()))


def _gru_proj_body(idx_ref, e_any, enc_ref, wih0_ref, whh0_ref, bi0_ref,
                   bh0_ref, wih1_ref, whh1_ref, bi1_ref, bh1_ref,
                   w1_ref, b1_ref, w2_ref, b2_ref, out_ref,
                   emb_ref, ys_ref, wih0t_ref, whh0t_ref, wih1t_ref,
                   whh1t_ref, sem):
    def row_copy(t, b):
        idx = idx_ref[b, t]
        return pltpu.make_async_copy(
            e_any.at[pl.ds(idx, 1), :],
            emb_ref.at[pl.ds(t * B + b, 1), :],
            sem)

    def issue_step(t):
        for b in range(B):
            row_copy(t, b).start()

    def wait_step(t):
        for b in range(B):
            row_copy(t, b).wait()

    for t in range(LOOKAHEAD):
        issue_step(t)

    wih0t_ref[...] = wih0_ref[...].T.astype(jnp.bfloat16)
    whh0t_ref[...] = whh0_ref[...].T.astype(jnp.bfloat16)
    wih1t_ref[...] = wih1_ref[...].T.astype(jnp.bfloat16)
    whh1t_ref[...] = whh1_ref[...].T.astype(jnp.bfloat16)

    def gru(x, h, wih, whh, bi, bh):
        gi = jnp.dot(x.astype(jnp.bfloat16), wih,
                     preferred_element_type=jnp.float32) + bi
        gh = jnp.dot(h.astype(jnp.bfloat16), whh,
                     preferred_element_type=jnp.float32) + bh
        i_r, i_z, i_n = gi[:, :HIDDEN], gi[:, HIDDEN:2 * HIDDEN], gi[:, 2 * HIDDEN:]
        h_r, h_z, h_n = gh[:, :HIDDEN], gh[:, HIDDEN:2 * HIDDEN], gh[:, 2 * HIDDEN:]
        r = jax.nn.sigmoid(i_r + h_r)
        z = jax.nn.sigmoid(i_z + h_z)
        n = jnp.tanh(i_n + r * h_n)
        return (1.0 - z) * n + z * h

    def step(t, carry):
        h0, h1 = carry

        @pl.when(t < T - LOOKAHEAD)
        def _():
            issue_step(t + LOOKAHEAD)

        wait_step(t)
        x = emb_ref[pl.ds(t * B, B), :]
        h0n = gru(x, h0, wih0t_ref[...], whh0t_ref[...],
                  bi0_ref[...].reshape(1, -1), bh0_ref[...].reshape(1, -1))
        h1n = gru(h0n, h1, wih1t_ref[...], whh1t_ref[...],
                  bi1_ref[...].reshape(1, -1), bh1_ref[...].reshape(1, -1))
        ys_ref[pl.ds(t * B, B), :] = h1n
        return (h0n, h1n)

    h0 = enc_ref[0]
    h1 = enc_ref[1]
    jax.lax.fori_loop(0, T, step, (h0, h1))
    ys = ys_ref[...]
    hid = jnp.tanh(jax.lax.dot_general(ys, w1_ref[...], _NT,
                                       preferred_element_type=jnp.float32)
                   + b1_ref[...].reshape(1, -1))
    out_ref[...] = jax.lax.dot_general(hid, w2_ref[...], _NT,
                                       preferred_element_type=jnp.float32) \
        + b2_ref[...].reshape(1, -1)


def _logits_body(proj_ref, e_ref, gb_ref, out_ref):
    res = jax.lax.dot_general(
        proj_ref[...], e_ref[...], _NT,
        preferred_element_type=jnp.float32) + gb_ref[...]
    out_ref[...] = res.reshape(T, B, res.shape[-1])


def kernel(encoding, trg, E, W_ih0, W_hh0, b_ih0, b_hh0, W_ih1, W_hh1,
           b_ih1, b_hh1, W1, b1, W2, b2, g_b):
    idx = trg.astype(jnp.int32)

    gru_weights = (
        encoding,
        W_ih0, W_hh0, b_ih0, b_hh0,
        W_ih1, W_hh1, b_ih1, b_hh1,
        W1, b1, W2, b2,
    )

    proj = pl.pallas_call(
        _gru_proj_body,
        in_specs=[pl.BlockSpec(memory_space=pltpu.SMEM),
                  pl.BlockSpec(memory_space=pltpu.MemorySpace.HBM)]
                 + [pl.BlockSpec(memory_space=pltpu.MemorySpace.VMEM)] * 13,
        out_shape=jax.ShapeDtypeStruct((BT, EMBED), jnp.float32),
        scratch_shapes=[pltpu.VMEM((BT, EMBED), jnp.float32),
                        pltpu.VMEM((BT, HIDDEN), jnp.float32),
                        pltpu.VMEM((EMBED, 3 * HIDDEN), jnp.bfloat16),
                        pltpu.VMEM((HIDDEN, 3 * HIDDEN), jnp.bfloat16),
                        pltpu.VMEM((HIDDEN, 3 * HIDDEN), jnp.bfloat16),
                        pltpu.VMEM((HIDDEN, 3 * HIDDEN), jnp.bfloat16),
                        pltpu.SemaphoreType.DMA],
    )(idx, E, *gru_weights)

    nv = pl.cdiv(VOCAB, VTILE)
    logits_tb = pl.pallas_call(
        _logits_body,
        grid=(nv,),
        in_specs=[
            pl.BlockSpec((BT, EMBED), lambda i: (0, 0)),
            pl.BlockSpec((VTILE, EMBED), lambda i: (i, 0)),
            pl.BlockSpec((1, VTILE), lambda i: (0, i)),
        ],
        out_specs=pl.BlockSpec((T, B, VTILE), lambda i: (0, 0, i)),
        out_shape=jax.ShapeDtypeStruct((T, B, VOCAB), jnp.float32),
        compiler_params=pltpu.CompilerParams(
            dimension_semantics=("arbitrary",)),
    )(proj, E, g_b.reshape(1, VOCAB))

    return logits_tb.transpose(1, 0, 2)


# speedup vs baseline: 1.3196x; 1.0151x over previous
import jax
import jax.numpy as jnp
from jax.experimental import pallas as pl
from jax.experimental.pallas import tpu as pltpu

VOCAB, EMBED, HIDDEN = 100000, 256, 512
B, T = 16, 20
BT = B * T
VTILE = 10240
LOOKAHEAD = 2

_NT = (((1,), (1,)), ((), ()))


def _gru_proj_body(idx_ref, e_any, enc_ref, wih0_ref, whh0_ref, bi0_ref,
                   bh0_ref, wih1_ref, whh1_ref, bi1_ref, bh1_ref,
                   w1_ref, b1_ref, w2_ref, b2_ref, out_ref,
                   emb_ref, ys_ref, wih0t_ref, whh0t_ref, wih1t_ref,
                   whh1t_ref, sem):
    def row_copy(t, b):
        idx = idx_ref[b, t]
        return pltpu.make_async_copy(
            e_any.at[pl.ds(idx, 1), :],
            emb_ref.at[pl.ds(t * B + b, 1), :],
            sem)

    def issue_step(t):
        for b in range(B):
            row_copy(t, b).start()

    def wait_step(t):
        for b in range(B):
            row_copy(t, b).wait()

    for t in range(LOOKAHEAD):
        issue_step(t)

    wih0t_ref[...] = wih0_ref[...].T.astype(jnp.bfloat16)
    whh0t_ref[...] = whh0_ref[...].T.astype(jnp.bfloat16)
    wih1t_ref[...] = wih1_ref[...].T.astype(jnp.bfloat16)
    whh1t_ref[...] = whh1_ref[...].T.astype(jnp.bfloat16)

    def gru(x, h, wih, whh, bi, bh):
        gi = jnp.dot(x.astype(jnp.bfloat16), wih,
                     preferred_element_type=jnp.float32) + bi
        gh = jnp.dot(h.astype(jnp.bfloat16), whh,
                     preferred_element_type=jnp.float32) + bh
        i_r, i_z, i_n = gi[:, :HIDDEN], gi[:, HIDDEN:2 * HIDDEN], gi[:, 2 * HIDDEN:]
        h_r, h_z, h_n = gh[:, :HIDDEN], gh[:, HIDDEN:2 * HIDDEN], gh[:, 2 * HIDDEN:]
        r = jax.nn.sigmoid(i_r + h_r)
        z = jax.nn.sigmoid(i_z + h_z)
        n = jnp.tanh(i_n + r * h_n)
        return (1.0 - z) * n + z * h

    bi0 = bi0_ref[...].reshape(1, -1)
    bh0 = bh0_ref[...].reshape(1, -1)
    bi1 = bi1_ref[...].reshape(1, -1)
    bh1 = bh1_ref[...].reshape(1, -1)
    h0 = enc_ref[0]
    h1 = enc_ref[1]
    for t in range(T):
        if t + LOOKAHEAD < T:
            issue_step(t + LOOKAHEAD)
        wait_step(t)
        x = emb_ref[t * B:(t + 1) * B, :]
        h0 = gru(x, h0, wih0t_ref[...], whh0t_ref[...], bi0, bh0)
        h1 = gru(h0, h1, wih1t_ref[...], whh1t_ref[...], bi1, bh1)
        ys_ref[t * B:(t + 1) * B, :] = h1
    ys = ys_ref[...]
    hid = jnp.tanh(jax.lax.dot_general(ys, w1_ref[...], _NT,
                                       preferred_element_type=jnp.float32)
                   + b1_ref[...].reshape(1, -1))
    out_ref[...] = jax.lax.dot_general(hid, w2_ref[...], _NT,
                                       preferred_element_type=jnp.float32) \
        + b2_ref[...].reshape(1, -1)


def _logits_body(proj_ref, e_ref, gb_ref, out_ref):
    res = jax.lax.dot_general(
        proj_ref[...], e_ref[...], _NT,
        preferred_element_type=jnp.float32) + gb_ref[...]
    out_ref[...] = res.reshape(T, B, res.shape[-1])


def kernel(encoding, trg, E, W_ih0, W_hh0, b_ih0, b_hh0, W_ih1, W_hh1,
           b_ih1, b_hh1, W1, b1, W2, b2, g_b):
    idx = trg.astype(jnp.int32)

    gru_weights = (
        encoding,
        W_ih0, W_hh0, b_ih0, b_hh0,
        W_ih1, W_hh1, b_ih1, b_hh1,
        W1, b1, W2, b2,
    )

    proj = pl.pallas_call(
        _gru_proj_body,
        in_specs=[pl.BlockSpec(memory_space=pltpu.SMEM),
                  pl.BlockSpec(memory_space=pltpu.MemorySpace.HBM)]
                 + [pl.BlockSpec(memory_space=pltpu.MemorySpace.VMEM)] * 13,
        out_shape=jax.ShapeDtypeStruct((BT, EMBED), jnp.float32),
        scratch_shapes=[pltpu.VMEM((BT, EMBED), jnp.float32),
                        pltpu.VMEM((BT, HIDDEN), jnp.float32),
                        pltpu.VMEM((EMBED, 3 * HIDDEN), jnp.bfloat16),
                        pltpu.VMEM((HIDDEN, 3 * HIDDEN), jnp.bfloat16),
                        pltpu.VMEM((HIDDEN, 3 * HIDDEN), jnp.bfloat16),
                        pltpu.VMEM((HIDDEN, 3 * HIDDEN), jnp.bfloat16),
                        pltpu.SemaphoreType.DMA],
    )(idx, E, *gru_weights)

    nv = pl.cdiv(VOCAB, VTILE)
    logits_tb = pl.pallas_call(
        _logits_body,
        grid=(nv,),
        in_specs=[
            pl.BlockSpec((BT, EMBED), lambda i: (0, 0)),
            pl.BlockSpec((VTILE, EMBED), lambda i: (i, 0)),
            pl.BlockSpec((1, VTILE), lambda i: (0, i)),
        ],
        out_specs=pl.BlockSpec((T, B, VTILE), lambda i: (0, 0, i)),
        out_shape=jax.ShapeDtypeStruct((T, B, VOCAB), jnp.float32),
        compiler_params=pltpu.CompilerParams(
            dimension_semantics=("arbitrary",)),
    )(proj, E, g_b.reshape(1, VOCAB))

    return logits_tb.transpose(1, 0, 2)


# speedup vs baseline: 1.3507x; 1.0236x over previous
import jax
import jax.numpy as jnp
from jax.experimental import pallas as pl
from jax.experimental.pallas import tpu as pltpu

VOCAB, EMBED, HIDDEN = 100000, 256, 512
B, T = 16, 20
BT = B * T
VTILE = 10240
LOOKAHEAD = 2

_NT = (((1,), (1,)), ((), ()))


def _gru_proj_body(idx_ref, e_any, enc_ref, wih0_ref, whh0_ref, bi0_ref,
                   bh0_ref, wih1_ref, whh1_ref, bi1_ref, bh1_ref,
                   w1_ref, b1_ref, w2_ref, b2_ref, out_ref,
                   emb_ref, ys_ref, wih0t_ref, whh0t_ref, wih1t_ref,
                   whh1t_ref, sem):
    def row_copy(t, b):
        idx = idx_ref[b, t]
        return pltpu.make_async_copy(
            e_any.at[pl.ds(idx, 1), :],
            emb_ref.at[pl.ds(t * B + b, 1), :],
            sem)

    def issue_step(t):
        for b in range(B):
            row_copy(t, b).start()

    def wait_step(t):
        for b in range(B):
            row_copy(t, b).wait()

    for t in range(LOOKAHEAD):
        issue_step(t)

    wih0t_ref[...] = wih0_ref[...].T.astype(jnp.bfloat16)
    whh0t_ref[...] = whh0_ref[...].T.astype(jnp.bfloat16)
    wih1t_ref[...] = wih1_ref[...].T.astype(jnp.bfloat16)
    whh1t_ref[...] = whh1_ref[...].T.astype(jnp.bfloat16)

    def gru(x, h, wih, whh, bi, bh):
        gi = jnp.dot(x.astype(jnp.bfloat16), wih,
                     preferred_element_type=jnp.float32) + bi
        gh = jnp.dot(h.astype(jnp.bfloat16), whh,
                     preferred_element_type=jnp.float32) + bh
        i_r, i_z, i_n = gi[:, :HIDDEN], gi[:, HIDDEN:2 * HIDDEN], gi[:, 2 * HIDDEN:]
        h_r, h_z, h_n = gh[:, :HIDDEN], gh[:, HIDDEN:2 * HIDDEN], gh[:, 2 * HIDDEN:]
        r = jax.nn.sigmoid(i_r + h_r)
        z = jax.nn.sigmoid(i_z + h_z)
        n = jnp.tanh(i_n + r * h_n)
        return (1.0 - z) * n + z * h

    bi0 = bi0_ref[...].reshape(1, -1)
    bh0 = bh0_ref[...].reshape(1, -1)
    bi1 = bi1_ref[...].reshape(1, -1)
    bh1 = bh1_ref[...].reshape(1, -1)
    h0 = enc_ref[0]
    h1 = enc_ref[1]
    for t in range(T):
        if t + LOOKAHEAD < T:
            issue_step(t + LOOKAHEAD)
        wait_step(t)
        x = emb_ref[t * B:(t + 1) * B, :]
        h0new = gru(x, h0, wih0t_ref[...], whh0t_ref[...], bi0, bh0)
        if t > 0:
            h1 = gru(h0, h1, wih1t_ref[...], whh1t_ref[...], bi1, bh1)
            ys_ref[(t - 1) * B:t * B, :] = h1
        h0 = h0new
    h1 = gru(h0, h1, wih1t_ref[...], whh1t_ref[...], bi1, bh1)
    ys_ref[(T - 1) * B:T * B, :] = h1
    ys = ys_ref[...]
    hid = jnp.tanh(jax.lax.dot_general(ys, w1_ref[...], _NT,
                                       preferred_element_type=jnp.float32)
                   + b1_ref[...].reshape(1, -1))
    out_ref[...] = jax.lax.dot_general(hid, w2_ref[...], _NT,
                                       preferred_element_type=jnp.float32) \
        + b2_ref[...].reshape(1, -1)


def _logits_body(proj_ref, e_ref, gb_ref, out_ref):
    res = jax.lax.dot_general(
        proj_ref[...], e_ref[...], _NT,
        preferred_element_type=jnp.float32) + gb_ref[...]
    out_ref[...] = res.reshape(T, B, res.shape[-1])


def kernel(encoding, trg, E, W_ih0, W_hh0, b_ih0, b_hh0, W_ih1, W_hh1,
           b_ih1, b_hh1, W1, b1, W2, b2, g_b):
    idx = trg.astype(jnp.int32)

    gru_weights = (
        encoding,
        W_ih0, W_hh0, b_ih0, b_hh0,
        W_ih1, W_hh1, b_ih1, b_hh1,
        W1, b1, W2, b2,
    )

    proj = pl.pallas_call(
        _gru_proj_body,
        in_specs=[pl.BlockSpec(memory_space=pltpu.SMEM),
                  pl.BlockSpec(memory_space=pltpu.MemorySpace.HBM)]
                 + [pl.BlockSpec(memory_space=pltpu.MemorySpace.VMEM)] * 13,
        out_shape=jax.ShapeDtypeStruct((BT, EMBED), jnp.float32),
        scratch_shapes=[pltpu.VMEM((BT, EMBED), jnp.float32),
                        pltpu.VMEM((BT, HIDDEN), jnp.float32),
                        pltpu.VMEM((EMBED, 3 * HIDDEN), jnp.bfloat16),
                        pltpu.VMEM((HIDDEN, 3 * HIDDEN), jnp.bfloat16),
                        pltpu.VMEM((HIDDEN, 3 * HIDDEN), jnp.bfloat16),
                        pltpu.VMEM((HIDDEN, 3 * HIDDEN), jnp.bfloat16),
                        pltpu.SemaphoreType.DMA],
    )(idx, E, *gru_weights)

    nv = pl.cdiv(VOCAB, VTILE)
    logits_tb = pl.pallas_call(
        _logits_body,
        grid=(nv,),
        in_specs=[
            pl.BlockSpec((BT, EMBED), lambda i: (0, 0)),
            pl.BlockSpec((VTILE, EMBED), lambda i: (i, 0)),
            pl.BlockSpec((1, VTILE), lambda i: (0, i)),
        ],
        out_specs=pl.BlockSpec((T, B, VTILE), lambda i: (0, 0, i)),
        out_shape=jax.ShapeDtypeStruct((T, B, VOCAB), jnp.float32),
        compiler_params=pltpu.CompilerParams(
            dimension_semantics=("arbitrary",)),
    )(proj, E, g_b.reshape(1, VOCAB))

    return logits_tb.transpose(1, 0, 2)
